# baseline (speedup 1.0000x reference)
import functools

import jax
import jax.numpy as jnp
from jax import lax
from jax.experimental import pallas as pl
from jax.experimental.pallas import tpu as pltpu
from jax.experimental.pallas import tpu_sc as plsc

B = 32
T = 512
N = 128
H = 256
C = 27
STEPS = 64


def _sc_gather(table, idx):
    info = plsc.get_sparse_core_info()
    nw = info.num_cores * info.num_subcores
    total = idx.shape[0]
    per_w = total // nw
    chunk = 256
    n_chunks = per_w // chunk
    h = table.shape[1]
    mesh = plsc.VectorSubcoreMesh(core_axis_name="c", subcore_axis_name="s")

    @functools.partial(
        pl.kernel,
        mesh=mesh,
        out_type=jax.ShapeDtypeStruct((total, h), jnp.float32),
        scratch_types=[
            pltpu.VMEM((chunk,), jnp.int32),
            pltpu.VMEM((chunk, h), jnp.float32),
            pltpu.SemaphoreType.DMA,
        ],
    )
    def k(table_hbm, idx_hbm, out_hbm, idx_v, rows_v, sem):
        wid = lax.axis_index("s") * info.num_cores + lax.axis_index("c")
        base = wid * per_w

        def body(i, carry):
            off = base + i * chunk
            pltpu.sync_copy(idx_hbm.at[pl.ds(off, chunk)], idx_v)
            pltpu.async_copy(table_hbm.at[idx_v], rows_v, sem).wait()
            pltpu.sync_copy(rows_v, out_hbm.at[pl.ds(off, chunk)])
            return carry

        lax.fori_loop(0, n_chunks, body, 0)

    return k(table, idx)


def _dot(a, b):
    return jnp.dot(a, b, preferred_element_type=jnp.float32)


def _bdot(a, b):
    return jnp.dot(a.astype(jnp.bfloat16), b.astype(jnp.bfloat16),
                   preferred_element_type=jnp.float32)


def _prologue_body(te_ref, st_ref, en_ref, wit_ref, bh_ref, out_ref):
    te = te_ref[0]
    post = lax.broadcasted_iota(jnp.int32, (T, N), 0)
    s_row = st_ref[0]
    e_row = en_ref[0]
    bf16 = jnp.bfloat16
    maskt = jnp.where((post >= s_row) & (post <= e_row), 1.0, 0.0).astype(bf16)
    cnt = jnp.maximum(e_row - s_row + 1, 1).astype(jnp.float32)
    node_t = lax.dot_general(
        te.astype(bf16), maskt, (((0,), (0,)), ((), ())),
        preferred_element_type=jnp.float32,
    ) / cnt
    out_ref[...] = _bdot(wit_ref[...], node_t) + bh_ref[...]


def _loop_body(msl_ref, xin_ref, sl_ref, exr_ref, ti_ref, fi_ref,
               wht_ref, wbt1_ref, bb_ref, wot_ref, bo_ref, out_ref):
    bf16 = jnp.bfloat16
    BN = B * N
    x_int = xin_ref[...]
    sl_row = sl_ref[...]
    wht = wht_ref[...].astype(bf16)
    wbt1 = wbt1_ref[...].astype(bf16)
    bb = bb_ref[...]

    lanes = lax.broadcasted_iota(jnp.int32, (N, N), 1)
    subl = lax.broadcasted_iota(jnp.int32, (N, N), 0)
    ident = jnp.where(lanes == subl, 1.0, 0.0)
    ohts = [jnp.where(lanes == ti_ref[b], 1.0, 0.0) for b in range(B)]
    ohfs = [jnp.where(lanes == fi_ref[b], 1.0, 0.0) for b in range(B)]

    lane_mod = lax.broadcasted_iota(jnp.int32, (1, BN), 1) % N
    p0 = jnp.where(lane_mod == 0, 1.0, 0.0)
    h0 = jnp.zeros((H, BN), bf16)
    ones_row = jnp.ones((1, BN), jnp.float32)

    def step(t, carry):
        p, htb = carry
        ht_new = jnp.tanh(x_int + jnp.dot(
            wht, htb, preferred_element_type=jnp.float32))
        ht1b = jnp.concatenate([ht_new, ones_row], axis=0).astype(bf16)
        lgt = jnp.dot(wbt1, ht1b, preferred_element_type=jnp.float32) + bb
        mx = jnp.max(lgt, axis=0, keepdims=True)
        elg = jnp.exp(lgt - mx)
        den = jnp.sum(elg, axis=0, keepdims=True)
        pt = p * (elg[0:1, :] / den)
        pf = p * (elg[1:2, :] / den)
        parts = []
        for b in range(B):
            s = slice(b * N, (b + 1) * N)
            ptc = jnp.sum(ident * pt[:, s], axis=1, keepdims=True)
            pfc = jnp.sum(ident * pf[:, s], axis=1, keepdims=True)
            m = (ohts[b] * ptc + ohfs[b] * pfc).astype(bf16)
            parts.append(jnp.dot(ht1b[:, s], m,
                                 preferred_element_type=jnp.float32))
        r = jnp.concatenate(parts, axis=1)
        p_next = r[H:H + 1, :]
        h_next = r[0:H, :] / jnp.maximum(p_next, 1e-6)
        act = t < sl_row
        p = jnp.where(act, p_next, p)
        htb = jnp.where(act, h_next.astype(bf16), htb)
        return (p, htb)

    msl = msl_ref[0]
    p, htb = lax.fori_loop(0, msl, step, (p0, h0))

    hx = htb.astype(jnp.float32) * exr_ref[...]
    cols = [jnp.sum(hx[:, b * N:(b + 1) * N], axis=1, keepdims=True)
            for b in range(B)]
    hsel = jnp.concatenate(cols, axis=1)
    out_ref[...] = _dot(wot_ref[...], hsel) + bo_ref[...]


def _tc_body(sl_ref, ex_ref, tet_ref, st_ref, en_ref, ti_ref, fi_ref,
             wit_ref, wht_ref, bh_ref, wbt_ref, bb_ref, wot_ref, bo_ref,
             out_ref):
    b = pl.program_id(0)
    tet = tet_ref[0]

    post = lax.broadcasted_iota(jnp.int32, (T, N), 0)
    s_row = st_ref[0]
    e_row = en_ref[0]
    maskt = jnp.where((post >= s_row) & (post <= e_row), 1.0, 0.0)
    cnt = jnp.maximum(jnp.sum(maskt, axis=0, keepdims=True), 1.0)
    node_t = _dot(tet, maskt) / cnt

    x_int = _dot(wit_ref[...], node_t) + bh_ref[...]

    lanes = lax.broadcasted_iota(jnp.int32, (N, N), 1)
    oht = jnp.where(lanes == ti_ref[0], 1.0, 0.0)
    ohf = jnp.where(lanes == fi_ref[0], 1.0, 0.0)

    lane_row = lax.broadcasted_iota(jnp.int32, (1, N), 1)
    p0 = jnp.where(lane_row == 0, 1.0, 0.0)
    h0 = jnp.zeros((H, N), jnp.float32)

    bf16 = jnp.bfloat16
    wht = wht_ref[...].astype(bf16)
    wbt = wbt_ref[...].astype(bf16)
    bb = bb_ref[...]
    ohtb = oht.astype(bf16)
    ohfb = ohf.astype(bf16)

    def step(_, carry):
        p, ht = carry
        ht_new = jnp.tanh(x_int + _bdot(wht, ht))
        lgt = _bdot(wbt, ht_new) + bb
        mx = jnp.max(lgt, axis=0, keepdims=True)
        elg = jnp.exp(lgt - mx)
        den = jnp.sum(elg, axis=0, keepdims=True)
        pt = p * (elg[0:1, :] / den)
        pf = p * (elg[1:2, :] / den)
        h_numt = _bdot(ht_new * pt, ohtb) + _bdot(ht_new * pf, ohfb)
        p_next = _dot(pt, oht) + _dot(pf, ohf)
        ht_next = h_numt / jnp.maximum(p_next, 1e-6)
        return (p_next, ht_next)

    sl = sl_ref[b]
    p, ht = lax.fori_loop(0, sl, step, (p0, h0))

    ex = ex_ref[b]
    oh_ex = jnp.where(lane_row == ex, 1.0, 0.0)
    hcol = jnp.sum(ht * oh_ex, axis=1, keepdims=True)
    out_ref[...] = (_dot(wot_ref[...], hcol) + bo_ref[...]).reshape(1, C, 1)


def _tc_main(tok_emb, starts, ends, tidx, fidx, step_limit, exit_index,
             w_in, w_h, b_h, w_branch, b_branch, w_out, b_out):
    row = lambda a: a.reshape(B, 1, N)
    BN = B * N
    x_int = pl.pallas_call(
        _prologue_body,
        grid=(B,),
        in_specs=[
            pl.BlockSpec((1, T, H), lambda b: (b, 0, 0)),
            pl.BlockSpec((1, 1, N), lambda b: (b, 0, 0)),
            pl.BlockSpec((1, 1, N), lambda b: (b, 0, 0)),
            pl.BlockSpec((H, H), lambda b: (0, 0)),
            pl.BlockSpec((H, 1), lambda b: (0, 0)),
        ],
        out_specs=pl.BlockSpec((H, N), lambda b: (0, b)),
        out_shape=jax.ShapeDtypeStruct((H, BN), jnp.float32),
    )(tok_emb, row(starts), row(ends), w_in.T, b_h.reshape(H, 1))

    sl_row = jnp.repeat(step_limit, N).reshape(1, BN)
    exr = (jnp.arange(N, dtype=jnp.int32)[None, :]
           == exit_index[:, None]).astype(jnp.float32).reshape(1, BN)
    msl = jnp.max(step_limit).reshape(1)

    grid_spec = pltpu.PrefetchScalarGridSpec(
        num_scalar_prefetch=1,
        grid=(1,),
        in_specs=[
            pl.BlockSpec((H, BN), lambda g, msl_r: (0, 0)),
            pl.BlockSpec((1, BN), lambda g, msl_r: (0, 0)),
            pl.BlockSpec((1, BN), lambda g, msl_r: (0, 0)),
            pl.BlockSpec((B, N, 1), lambda g, msl_r: (0, 0, 0)),
            pl.BlockSpec((B, N, 1), lambda g, msl_r: (0, 0, 0)),
            pl.BlockSpec((H, H), lambda g, msl_r: (0, 0)),
            pl.BlockSpec((2, H + 1), lambda g, msl_r: (0, 0)),
            pl.BlockSpec((2, 1), lambda g, msl_r: (0, 0)),
            pl.BlockSpec((C, H), lambda g, msl_r: (0, 0)),
            pl.BlockSpec((C, 1), lambda g, msl_r: (0, 0)),
        ],
        out_specs=pl.BlockSpec((C, B), lambda g, msl_r: (0, 0)),
    )
    out = pl.pallas_call(
        _loop_body,
        grid_spec=grid_spec,
        out_shape=jax.ShapeDtypeStruct((C, B), jnp.float32),
    )(
        msl, x_int, sl_row, exr,
        tidx.reshape(B, N, 1), fidx.reshape(B, N, 1),
        w_h.T,
        jnp.concatenate([w_branch.T, jnp.zeros((2, 1), jnp.float32)], axis=1),
        b_branch.reshape(2, 1),
        w_out.T, b_out.reshape(C, 1),
    )
    return out.T


def _tc_main_old(tok_embt, starts, ends, tidx, fidx, step_limit, exit_index,
                 w_in, w_h, b_h, w_branch, b_branch, w_out, b_out):
    col = lambda a: a.reshape(B, N, 1)
    row = lambda a: a.reshape(B, 1, N)
    grid_spec = pltpu.PrefetchScalarGridSpec(
        num_scalar_prefetch=2,
        grid=(B,),
        in_specs=[
            pl.BlockSpec((1, H, T), lambda b, sl, ex: (b, 0, 0)),
            pl.BlockSpec((1, 1, N), lambda b, sl, ex: (b, 0, 0)),
            pl.BlockSpec((1, 1, N), lambda b, sl, ex: (b, 0, 0)),
            pl.BlockSpec((1, N, 1), lambda b, sl, ex: (b, 0, 0)),
            pl.BlockSpec((1, N, 1), lambda b, sl, ex: (b, 0, 0)),
            pl.BlockSpec((H, H), lambda b, sl, ex: (0, 0)),
            pl.BlockSpec((H, H), lambda b, sl, ex: (0, 0)),
            pl.BlockSpec((H, 1), lambda b, sl, ex: (0, 0)),
            pl.BlockSpec((2, H), lambda b, sl, ex: (0, 0)),
            pl.BlockSpec((2, 1), lambda b, sl, ex: (0, 0)),
            pl.BlockSpec((C, H), lambda b, sl, ex: (0, 0)),
            pl.BlockSpec((C, 1), lambda b, sl, ex: (0, 0)),
        ],
        out_specs=pl.BlockSpec((1, C, 1), lambda b, sl, ex: (b, 0, 0)),
    )
    out = pl.pallas_call(
        _tc_body,
        grid_spec=grid_spec,
        out_shape=jax.ShapeDtypeStruct((B, C, 1), jnp.float32),
    )(
        step_limit, exit_index,
        tok_embt, row(starts), row(ends), col(tidx), col(fidx),
        w_in.T, w_h.T, b_h.reshape(H, 1), w_branch.T,
        b_branch.reshape(2, 1), w_out.T, b_out.reshape(C, 1),
    )
    return out.reshape(B, C)


def kernel(tokens, node_token_span_starts, node_token_span_ends,
           edge_sources, edge_dests, edge_types, true_branch_nodes,
           false_branch_nodes, exit_index, step_limit, embed, W_in, W_h,
           b_h, W_branch, b_branch, W_out, b_out):
    i32 = jnp.int32
    tokens = tokens.astype(i32)
    tok_emb = _sc_gather(embed, tokens.reshape(-1)).reshape(B, T, H)
    return _tc_main(
        tok_emb,
        node_token_span_starts.astype(i32),
        node_token_span_ends.astype(i32),
        true_branch_nodes.astype(i32),
        false_branch_nodes.astype(i32),
        step_limit.astype(i32),
        exit_index.astype(i32),
        W_in, W_h, b_h, W_branch, b_branch, W_out, b_out,
    )


# speedup vs baseline: 1.0760x; 1.0760x over previous
import functools

import jax
import jax.numpy as jnp
from jax import lax
from jax.experimental import pallas as pl
from jax.experimental.pallas import tpu as pltpu
from jax.experimental.pallas import tpu_sc as plsc

B = 32
T = 512
N = 128
H = 256
C = 27
STEPS = 64


def _sc_gather(table, idx):
    info = plsc.get_sparse_core_info()
    nw = info.num_cores * info.num_subcores
    total = idx.shape[0]
    per_w = total // nw
    chunk = 256
    n_chunks = per_w // chunk
    h = table.shape[1]
    mesh = plsc.VectorSubcoreMesh(core_axis_name="c", subcore_axis_name="s")

    @functools.partial(
        pl.kernel,
        mesh=mesh,
        out_type=jax.ShapeDtypeStruct((total, h), jnp.float32),
        scratch_types=[
            pltpu.VMEM((chunk,), jnp.int32),
            pltpu.VMEM((chunk, h), jnp.float32),
            pltpu.SemaphoreType.DMA,
        ],
    )
    def k(table_hbm, idx_hbm, out_hbm, idx_v, rows_v, sem):
        wid = lax.axis_index("s") * info.num_cores + lax.axis_index("c")
        base = wid * per_w

        def body(i, carry):
            off = base + i * chunk
            pltpu.sync_copy(idx_hbm.at[pl.ds(off, chunk)], idx_v)
            pltpu.async_copy(table_hbm.at[idx_v], rows_v, sem).wait()
            pltpu.sync_copy(rows_v, out_hbm.at[pl.ds(off, chunk)])
            return carry

        lax.fori_loop(0, n_chunks, body, 0)

    return k(table, idx)


def _dot(a, b):
    return jnp.dot(a, b, preferred_element_type=jnp.float32)


def _bdot(a, b):
    return jnp.dot(a.astype(jnp.bfloat16), b.astype(jnp.bfloat16),
                   preferred_element_type=jnp.float32)


def _prologue_body(te_ref, st_ref, en_ref, wit_ref, bh_ref, out_ref):
    te = te_ref[0]
    post = lax.broadcasted_iota(jnp.int32, (T, N), 0)
    s_row = st_ref[0]
    e_row = en_ref[0]
    bf16 = jnp.bfloat16
    maskt = jnp.where((post >= s_row) & (post <= e_row), 1.0, 0.0).astype(bf16)
    cnt = jnp.maximum(e_row - s_row + 1, 1).astype(jnp.float32)
    node_t = lax.dot_general(
        te.astype(bf16), maskt, (((0,), (0,)), ((), ())),
        preferred_element_type=jnp.float32,
    ) / cnt
    out_ref[...] = _bdot(wit_ref[...], node_t) + bh_ref[...]


def _fused_body(msl_ref, te_ref, st_ref, en_ref, sl_ref, exr_ref,
                ti_ref, fi_ref, wit_ref, bh_ref,
                wht_ref, wbt1_ref, bb_ref, wot_ref, bo_ref, out_ref):
    bf16 = jnp.bfloat16
    BN = B * N

    post = lax.broadcasted_iota(jnp.int32, (T, N), 0)
    witb = wit_ref[...].astype(bf16)
    xparts = []
    for b in range(B):
        s_row = st_ref[b]
        e_row = en_ref[b]
        maskt = jnp.where((post >= s_row) & (post <= e_row),
                          1.0, 0.0).astype(bf16)
        cnt = jnp.maximum(e_row - s_row + 1, 1).astype(jnp.float32)
        node_t = lax.dot_general(
            te_ref[b].astype(bf16), maskt, (((0,), (0,)), ((), ())),
            preferred_element_type=jnp.float32,
        ) / cnt
        xparts.append(
            jnp.dot(witb, node_t.astype(bf16),
                    preferred_element_type=jnp.float32))
    x_int = jnp.concatenate(xparts, axis=1) + bh_ref[...]
    sl_row = sl_ref[...]
    wht = wht_ref[...].astype(bf16)
    wbt1 = wbt1_ref[...].astype(bf16)
    bb = bb_ref[...]

    lanes = lax.broadcasted_iota(jnp.int32, (N, N), 1)
    subl = lax.broadcasted_iota(jnp.int32, (N, N), 0)
    ident = jnp.where(lanes == subl, 1.0, 0.0)
    ohts = [jnp.where(lanes == ti_ref[b], 1.0, 0.0) for b in range(B)]
    ohfs = [jnp.where(lanes == fi_ref[b], 1.0, 0.0) for b in range(B)]

    lane_mod = lax.broadcasted_iota(jnp.int32, (1, BN), 1) % N
    p0 = jnp.where(lane_mod == 0, 1.0, 0.0)
    h0 = jnp.zeros((H, BN), bf16)
    ones_row = jnp.ones((1, BN), jnp.float32)

    def step(t, carry):
        p, htb = carry
        ht_new = jnp.tanh(x_int + jnp.dot(
            wht, htb, preferred_element_type=jnp.float32))
        ht1b = jnp.concatenate([ht_new, ones_row], axis=0).astype(bf16)
        lgt = jnp.dot(wbt1, ht1b, preferred_element_type=jnp.float32) + bb
        mx = jnp.max(lgt, axis=0, keepdims=True)
        elg = jnp.exp(lgt - mx)
        den = jnp.sum(elg, axis=0, keepdims=True)
        pt = p * (elg[0:1, :] / den)
        pf = p * (elg[1:2, :] / den)
        parts = []
        for b in range(B):
            s = slice(b * N, (b + 1) * N)
            ptc = jnp.sum(ident * pt[:, s], axis=1, keepdims=True)
            pfc = jnp.sum(ident * pf[:, s], axis=1, keepdims=True)
            m = (ohts[b] * ptc + ohfs[b] * pfc).astype(bf16)
            parts.append(jnp.dot(ht1b[:, s], m,
                                 preferred_element_type=jnp.float32))
        r = jnp.concatenate(parts, axis=1)
        p_next = r[H:H + 1, :]
        h_next = r[0:H, :] / jnp.maximum(p_next, 1e-6)
        act = t < sl_row
        p = jnp.where(act, p_next, p)
        htb = jnp.where(act, h_next.astype(bf16), htb)
        return (p, htb)

    msl = msl_ref[0]
    p, htb = lax.fori_loop(0, msl, step, (p0, h0))

    hx = htb.astype(jnp.float32) * exr_ref[...]
    cols = [jnp.sum(hx[:, b * N:(b + 1) * N], axis=1, keepdims=True)
            for b in range(B)]
    hsel = jnp.concatenate(cols, axis=1)
    out_ref[...] = _dot(wot_ref[...], hsel) + bo_ref[...]


def _tc_body(sl_ref, ex_ref, tet_ref, st_ref, en_ref, ti_ref, fi_ref,
             wit_ref, wht_ref, bh_ref, wbt_ref, bb_ref, wot_ref, bo_ref,
             out_ref):
    b = pl.program_id(0)
    tet = tet_ref[0]

    post = lax.broadcasted_iota(jnp.int32, (T, N), 0)
    s_row = st_ref[0]
    e_row = en_ref[0]
    maskt = jnp.where((post >= s_row) & (post <= e_row), 1.0, 0.0)
    cnt = jnp.maximum(jnp.sum(maskt, axis=0, keepdims=True), 1.0)
    node_t = _dot(tet, maskt) / cnt

    x_int = _dot(wit_ref[...], node_t) + bh_ref[...]

    lanes = lax.broadcasted_iota(jnp.int32, (N, N), 1)
    oht = jnp.where(lanes == ti_ref[0], 1.0, 0.0)
    ohf = jnp.where(lanes == fi_ref[0], 1.0, 0.0)

    lane_row = lax.broadcasted_iota(jnp.int32, (1, N), 1)
    p0 = jnp.where(lane_row == 0, 1.0, 0.0)
    h0 = jnp.zeros((H, N), jnp.float32)

    bf16 = jnp.bfloat16
    wht = wht_ref[...].astype(bf16)
    wbt = wbt_ref[...].astype(bf16)
    bb = bb_ref[...]
    ohtb = oht.astype(bf16)
    ohfb = ohf.astype(bf16)

    def step(_, carry):
        p, ht = carry
        ht_new = jnp.tanh(x_int + _bdot(wht, ht))
        lgt = _bdot(wbt, ht_new) + bb
        mx = jnp.max(lgt, axis=0, keepdims=True)
        elg = jnp.exp(lgt - mx)
        den = jnp.sum(elg, axis=0, keepdims=True)
        pt = p * (elg[0:1, :] / den)
        pf = p * (elg[1:2, :] / den)
        h_numt = _bdot(ht_new * pt, ohtb) + _bdot(ht_new * pf, ohfb)
        p_next = _dot(pt, oht) + _dot(pf, ohf)
        ht_next = h_numt / jnp.maximum(p_next, 1e-6)
        return (p_next, ht_next)

    sl = sl_ref[b]
    p, ht = lax.fori_loop(0, sl, step, (p0, h0))

    ex = ex_ref[b]
    oh_ex = jnp.where(lane_row == ex, 1.0, 0.0)
    hcol = jnp.sum(ht * oh_ex, axis=1, keepdims=True)
    out_ref[...] = (_dot(wot_ref[...], hcol) + bo_ref[...]).reshape(1, C, 1)


def _tc_main(tok_emb, starts, ends, tidx, fidx, step_limit, exit_index,
             w_in, w_h, b_h, w_branch, b_branch, w_out, b_out):
    row = lambda a: a.reshape(B, 1, N)
    BN = B * N

    sl_row = jnp.repeat(step_limit, N).reshape(1, BN)
    exr = (jnp.arange(N, dtype=jnp.int32)[None, :]
           == exit_index[:, None]).astype(jnp.float32).reshape(1, BN)
    msl = jnp.max(step_limit).reshape(1)

    grid_spec = pltpu.PrefetchScalarGridSpec(
        num_scalar_prefetch=1,
        grid=(1,),
        in_specs=[
            pl.BlockSpec((B, T, H), lambda g, msl_r: (0, 0, 0)),
            pl.BlockSpec((B, 1, N), lambda g, msl_r: (0, 0, 0)),
            pl.BlockSpec((B, 1, N), lambda g, msl_r: (0, 0, 0)),
            pl.BlockSpec((1, BN), lambda g, msl_r: (0, 0)),
            pl.BlockSpec((1, BN), lambda g, msl_r: (0, 0)),
            pl.BlockSpec((B, N, 1), lambda g, msl_r: (0, 0, 0)),
            pl.BlockSpec((B, N, 1), lambda g, msl_r: (0, 0, 0)),
            pl.BlockSpec((H, H), lambda g, msl_r: (0, 0)),
            pl.BlockSpec((H, 1), lambda g, msl_r: (0, 0)),
            pl.BlockSpec((H, H), lambda g, msl_r: (0, 0)),
            pl.BlockSpec((2, H + 1), lambda g, msl_r: (0, 0)),
            pl.BlockSpec((2, 1), lambda g, msl_r: (0, 0)),
            pl.BlockSpec((C, H), lambda g, msl_r: (0, 0)),
            pl.BlockSpec((C, 1), lambda g, msl_r: (0, 0)),
        ],
        out_specs=pl.BlockSpec((C, B), lambda g, msl_r: (0, 0)),
    )
    out = pl.pallas_call(
        _fused_body,
        grid_spec=grid_spec,
        out_shape=jax.ShapeDtypeStruct((C, B), jnp.float32),
    )(
        msl, tok_emb, row(starts), row(ends), sl_row, exr,
        tidx.reshape(B, N, 1), fidx.reshape(B, N, 1),
        w_in.T, b_h.reshape(H, 1),
        w_h.T,
        jnp.concatenate([w_branch.T, jnp.zeros((2, 1), jnp.float32)], axis=1),
        b_branch.reshape(2, 1),
        w_out.T, b_out.reshape(C, 1),
    )
    return out.T


def _tc_main_old(tok_embt, starts, ends, tidx, fidx, step_limit, exit_index,
                 w_in, w_h, b_h, w_branch, b_branch, w_out, b_out):
    col = lambda a: a.reshape(B, N, 1)
    row = lambda a: a.reshape(B, 1, N)
    grid_spec = pltpu.PrefetchScalarGridSpec(
        num_scalar_prefetch=2,
        grid=(B,),
        in_specs=[
            pl.BlockSpec((1, H, T), lambda b, sl, ex: (b, 0, 0)),
            pl.BlockSpec((1, 1, N), lambda b, sl, ex: (b, 0, 0)),
            pl.BlockSpec((1, 1, N), lambda b, sl, ex: (b, 0, 0)),
            pl.BlockSpec((1, N, 1), lambda b, sl, ex: (b, 0, 0)),
            pl.BlockSpec((1, N, 1), lambda b, sl, ex: (b, 0, 0)),
            pl.BlockSpec((H, H), lambda b, sl, ex: (0, 0)),
            pl.BlockSpec((H, H), lambda b, sl, ex: (0, 0)),
            pl.BlockSpec((H, 1), lambda b, sl, ex: (0, 0)),
            pl.BlockSpec((2, H), lambda b, sl, ex: (0, 0)),
            pl.BlockSpec((2, 1), lambda b, sl, ex: (0, 0)),
            pl.BlockSpec((C, H), lambda b, sl, ex: (0, 0)),
            pl.BlockSpec((C, 1), lambda b, sl, ex: (0, 0)),
        ],
        out_specs=pl.BlockSpec((1, C, 1), lambda b, sl, ex: (b, 0, 0)),
    )
    out = pl.pallas_call(
        _tc_body,
        grid_spec=grid_spec,
        out_shape=jax.ShapeDtypeStruct((B, C, 1), jnp.float32),
    )(
        step_limit, exit_index,
        tok_embt, row(starts), row(ends), col(tidx), col(fidx),
        w_in.T, w_h.T, b_h.reshape(H, 1), w_branch.T,
        b_branch.reshape(2, 1), w_out.T, b_out.reshape(C, 1),
    )
    return out.reshape(B, C)


def kernel(tokens, node_token_span_starts, node_token_span_ends,
           edge_sources, edge_dests, edge_types, true_branch_nodes,
           false_branch_nodes, exit_index, step_limit, embed, W_in, W_h,
           b_h, W_branch, b_branch, W_out, b_out):
    i32 = jnp.int32
    tokens = tokens.astype(i32)
    tok_emb = _sc_gather(embed, tokens.reshape(-1)).reshape(B, T, H)
    return _tc_main(
        tok_emb,
        node_token_span_starts.astype(i32),
        node_token_span_ends.astype(i32),
        true_branch_nodes.astype(i32),
        false_branch_nodes.astype(i32),
        step_limit.astype(i32),
        exit_index.astype(i32),
        W_in, W_h, b_h, W_branch, b_branch, W_out, b_out,
    )


# speedup vs baseline: 1.0783x; 1.0021x over previous
import functools

import jax
import jax.numpy as jnp
from jax import lax
from jax.experimental import pallas as pl
from jax.experimental.pallas import tpu as pltpu
from jax.experimental.pallas import tpu_sc as plsc

B = 32
T = 512
N = 128
H = 256
C = 27
STEPS = 64


def _sc_gather(table, idx):
    info = plsc.get_sparse_core_info()
    nw = info.num_cores * info.num_subcores
    total = idx.shape[0]
    per_w = total // nw
    chunk = 128
    n_chunks = per_w // chunk
    h = table.shape[1]
    mesh = plsc.VectorSubcoreMesh(core_axis_name="c", subcore_axis_name="s")
    idx2 = idx.reshape(total // chunk, chunk)

    @functools.partial(
        pl.kernel,
        mesh=mesh,
        out_type=jax.ShapeDtypeStruct((total, h), jnp.float32),
        scratch_types=[
            pltpu.VMEM((n_chunks, chunk), jnp.int32),
            pltpu.VMEM((chunk, h), jnp.float32),
            pltpu.VMEM((chunk, h), jnp.float32),
            pltpu.SemaphoreType.DMA,
            pltpu.SemaphoreType.DMA,
            pltpu.SemaphoreType.DMA,
            pltpu.SemaphoreType.DMA,
        ],
    )
    def k(table_hbm, idx_hbm, out_hbm, idx_v, r0, r1, g0, g1, w0, w1):
        wid = lax.axis_index("s") * info.num_cores + lax.axis_index("c")
        base = wid * per_w
        pltpu.sync_copy(idx_hbm.at[pl.ds(wid * n_chunks, n_chunks)], idx_v)
        bufs = [r0, r1]
        gsems = [g0, g1]
        wsems = [w0, w1]
        gd = [None, None]
        wd = [None, None]
        for c in range(2):
            gd[c] = pltpu.async_copy(
                table_hbm.at[idx_v.at[c]], bufs[c], gsems[c])
        for c in range(n_chunks):
            i = c % 2
            gd[i].wait()
            wd[i] = pltpu.async_copy(
                bufs[i], out_hbm.at[pl.ds(base + c * chunk, chunk)], wsems[i])
            if c + 2 < n_chunks:
                wd[i].wait()
                gd[i] = pltpu.async_copy(
                    table_hbm.at[idx_v.at[c + 2]], bufs[i], gsems[i])
                wd[i] = None
        for i in range(2):
            if wd[i] is not None:
                wd[i].wait()

    return k(table, idx2)


def _dot(a, b):
    return jnp.dot(a, b, preferred_element_type=jnp.float32)


def _bdot(a, b):
    return jnp.dot(a.astype(jnp.bfloat16), b.astype(jnp.bfloat16),
                   preferred_element_type=jnp.float32)


def _prologue_body(te_ref, st_ref, en_ref, wit_ref, bh_ref, out_ref):
    te = te_ref[0]
    post = lax.broadcasted_iota(jnp.int32, (T, N), 0)
    s_row = st_ref[0]
    e_row = en_ref[0]
    bf16 = jnp.bfloat16
    maskt = jnp.where((post >= s_row) & (post <= e_row), 1.0, 0.0).astype(bf16)
    cnt = jnp.maximum(e_row - s_row + 1, 1).astype(jnp.float32)
    node_t = lax.dot_general(
        te.astype(bf16), maskt, (((0,), (0,)), ((), ())),
        preferred_element_type=jnp.float32,
    ) / cnt
    out_ref[...] = _bdot(wit_ref[...], node_t) + bh_ref[...]


def _fused_body(msl_ref, te_ref, st_ref, en_ref, sl_ref, exr_ref,
                ti_ref, fi_ref, wit_ref, bh_ref,
                wht_ref, wbt1_ref, bb_ref, wot_ref, bo_ref, out_ref):
    bf16 = jnp.bfloat16
    BN = B * N

    post = lax.broadcasted_iota(jnp.int32, (T, N), 0)
    witb = wit_ref[...].astype(bf16)
    xparts = []
    for b in range(B):
        s_row = st_ref[b]
        e_row = en_ref[b]
        maskt = jnp.where((post >= s_row) & (post <= e_row),
                          1.0, 0.0).astype(bf16)
        cnt = jnp.maximum(e_row - s_row + 1, 1).astype(jnp.float32)
        node_t = lax.dot_general(
            te_ref[b].astype(bf16), maskt, (((0,), (0,)), ((), ())),
            preferred_element_type=jnp.float32,
        ) / cnt
        xparts.append(
            jnp.dot(witb, node_t.astype(bf16),
                    preferred_element_type=jnp.float32))
    x_int = jnp.concatenate(xparts, axis=1) + bh_ref[...]
    sl_row = sl_ref[...]
    wht = wht_ref[...].astype(bf16)
    wbt1 = wbt1_ref[...].astype(bf16)
    bb = bb_ref[...]

    lanes = lax.broadcasted_iota(jnp.int32, (N, N), 1)
    subl = lax.broadcasted_iota(jnp.int32, (N, N), 0)
    ident = jnp.where(lanes == subl, 1.0, 0.0)
    ohts = [jnp.where(lanes == ti_ref[b], 1.0, 0.0) for b in range(B)]
    ohfs = [jnp.where(lanes == fi_ref[b], 1.0, 0.0) for b in range(B)]

    lane_mod = lax.broadcasted_iota(jnp.int32, (1, BN), 1) % N
    p0 = jnp.where(lane_mod == 0, 1.0, 0.0)
    h0 = jnp.zeros((H, BN), bf16)
    ones_row = jnp.ones((1, BN), jnp.float32)

    def step(t, carry):
        p, htb = carry
        ht_new = jnp.tanh(x_int + jnp.dot(
            wht, htb, preferred_element_type=jnp.float32))
        ht1b = jnp.concatenate([ht_new, ones_row], axis=0).astype(bf16)
        lgt = jnp.dot(wbt1, ht1b, preferred_element_type=jnp.float32) + bb
        mx = jnp.max(lgt, axis=0, keepdims=True)
        elg = jnp.exp(lgt - mx)
        den = jnp.sum(elg, axis=0, keepdims=True)
        pt = p * (elg[0:1, :] / den)
        pf = p * (elg[1:2, :] / den)
        parts = []
        for b in range(B):
            s = slice(b * N, (b + 1) * N)
            ptc = jnp.sum(ident * pt[:, s], axis=1, keepdims=True)
            pfc = jnp.sum(ident * pf[:, s], axis=1, keepdims=True)
            m = (ohts[b] * ptc + ohfs[b] * pfc).astype(bf16)
            parts.append(jnp.dot(ht1b[:, s], m,
                                 preferred_element_type=jnp.float32))
        r = jnp.concatenate(parts, axis=1)
        p_next = r[H:H + 1, :]
        h_next = r[0:H, :] / jnp.maximum(p_next, 1e-6)
        act = t < sl_row
        p = jnp.where(act, p_next, p)
        htb = jnp.where(act, h_next.astype(bf16), htb)
        return (p, htb)

    msl = msl_ref[0]
    p, htb = lax.fori_loop(0, msl, step, (p0, h0))

    hx = htb.astype(jnp.float32) * exr_ref[...]
    cols = [jnp.sum(hx[:, b * N:(b + 1) * N], axis=1, keepdims=True)
            for b in range(B)]
    hsel = jnp.concatenate(cols, axis=1)
    out_ref[...] = _dot(wot_ref[...], hsel) + bo_ref[...]


def _tc_body(sl_ref, ex_ref, tet_ref, st_ref, en_ref, ti_ref, fi_ref,
             wit_ref, wht_ref, bh_ref, wbt_ref, bb_ref, wot_ref, bo_ref,
             out_ref):
    b = pl.program_id(0)
    tet = tet_ref[0]

    post = lax.broadcasted_iota(jnp.int32, (T, N), 0)
    s_row = st_ref[0]
    e_row = en_ref[0]
    maskt = jnp.where((post >= s_row) & (post <= e_row), 1.0, 0.0)
    cnt = jnp.maximum(jnp.sum(maskt, axis=0, keepdims=True), 1.0)
    node_t = _dot(tet, maskt) / cnt

    x_int = _dot(wit_ref[...], node_t) + bh_ref[...]

    lanes = lax.broadcasted_iota(jnp.int32, (N, N), 1)
    oht = jnp.where(lanes == ti_ref[0], 1.0, 0.0)
    ohf = jnp.where(lanes == fi_ref[0], 1.0, 0.0)

    lane_row = lax.broadcasted_iota(jnp.int32, (1, N), 1)
    p0 = jnp.where(lane_row == 0, 1.0, 0.0)
    h0 = jnp.zeros((H, N), jnp.float32)

    bf16 = jnp.bfloat16
    wht = wht_ref[...].astype(bf16)
    wbt = wbt_ref[...].astype(bf16)
    bb = bb_ref[...]
    ohtb = oht.astype(bf16)
    ohfb = ohf.astype(bf16)

    def step(_, carry):
        p, ht = carry
        ht_new = jnp.tanh(x_int + _bdot(wht, ht))
        lgt = _bdot(wbt, ht_new) + bb
        mx = jnp.max(lgt, axis=0, keepdims=True)
        elg = jnp.exp(lgt - mx)
        den = jnp.sum(elg, axis=0, keepdims=True)
        pt = p * (elg[0:1, :] / den)
        pf = p * (elg[1:2, :] / den)
        h_numt = _bdot(ht_new * pt, ohtb) + _bdot(ht_new * pf, ohfb)
        p_next = _dot(pt, oht) + _dot(pf, ohf)
        ht_next = h_numt / jnp.maximum(p_next, 1e-6)
        return (p_next, ht_next)

    sl = sl_ref[b]
    p, ht = lax.fori_loop(0, sl, step, (p0, h0))

    ex = ex_ref[b]
    oh_ex = jnp.where(lane_row == ex, 1.0, 0.0)
    hcol = jnp.sum(ht * oh_ex, axis=1, keepdims=True)
    out_ref[...] = (_dot(wot_ref[...], hcol) + bo_ref[...]).reshape(1, C, 1)


def _tc_main(tok_emb, starts, ends, tidx, fidx, step_limit, exit_index,
             w_in, w_h, b_h, w_branch, b_branch, w_out, b_out):
    row = lambda a: a.reshape(B, 1, N)
    BN = B * N

    sl_row = jnp.repeat(step_limit, N).reshape(1, BN)
    exr = (jnp.arange(N, dtype=jnp.int32)[None, :]
           == exit_index[:, None]).astype(jnp.float32).reshape(1, BN)
    msl = jnp.max(step_limit).reshape(1)

    grid_spec = pltpu.PrefetchScalarGridSpec(
        num_scalar_prefetch=1,
        grid=(1,),
        in_specs=[
            pl.BlockSpec((B, T, H), lambda g, msl_r: (0, 0, 0)),
            pl.BlockSpec((B, 1, N), lambda g, msl_r: (0, 0, 0)),
            pl.BlockSpec((B, 1, N), lambda g, msl_r: (0, 0, 0)),
            pl.BlockSpec((1, BN), lambda g, msl_r: (0, 0)),
            pl.BlockSpec((1, BN), lambda g, msl_r: (0, 0)),
            pl.BlockSpec((B, N, 1), lambda g, msl_r: (0, 0, 0)),
            pl.BlockSpec((B, N, 1), lambda g, msl_r: (0, 0, 0)),
            pl.BlockSpec((H, H), lambda g, msl_r: (0, 0)),
            pl.BlockSpec((H, 1), lambda g, msl_r: (0, 0)),
            pl.BlockSpec((H, H), lambda g, msl_r: (0, 0)),
            pl.BlockSpec((2, H + 1), lambda g, msl_r: (0, 0)),
            pl.BlockSpec((2, 1), lambda g, msl_r: (0, 0)),
            pl.BlockSpec((C, H), lambda g, msl_r: (0, 0)),
            pl.BlockSpec((C, 1), lambda g, msl_r: (0, 0)),
        ],
        out_specs=pl.BlockSpec((C, B), lambda g, msl_r: (0, 0)),
    )
    out = pl.pallas_call(
        _fused_body,
        grid_spec=grid_spec,
        out_shape=jax.ShapeDtypeStruct((C, B), jnp.float32),
    )(
        msl, tok_emb, row(starts), row(ends), sl_row, exr,
        tidx.reshape(B, N, 1), fidx.reshape(B, N, 1),
        w_in.T, b_h.reshape(H, 1),
        w_h.T,
        jnp.concatenate([w_branch.T, jnp.zeros((2, 1), jnp.float32)], axis=1),
        b_branch.reshape(2, 1),
        w_out.T, b_out.reshape(C, 1),
    )
    return out.T


def _tc_main_old(tok_embt, starts, ends, tidx, fidx, step_limit, exit_index,
                 w_in, w_h, b_h, w_branch, b_branch, w_out, b_out):
    col = lambda a: a.reshape(B, N, 1)
    row = lambda a: a.reshape(B, 1, N)
    grid_spec = pltpu.PrefetchScalarGridSpec(
        num_scalar_prefetch=2,
        grid=(B,),
        in_specs=[
            pl.BlockSpec((1, H, T), lambda b, sl, ex: (b, 0, 0)),
            pl.BlockSpec((1, 1, N), lambda b, sl, ex: (b, 0, 0)),
            pl.BlockSpec((1, 1, N), lambda b, sl, ex: (b, 0, 0)),
            pl.BlockSpec((1, N, 1), lambda b, sl, ex: (b, 0, 0)),
            pl.BlockSpec((1, N, 1), lambda b, sl, ex: (b, 0, 0)),
            pl.BlockSpec((H, H), lambda b, sl, ex: (0, 0)),
            pl.BlockSpec((H, H), lambda b, sl, ex: (0, 0)),
            pl.BlockSpec((H, 1), lambda b, sl, ex: (0, 0)),
            pl.BlockSpec((2, H), lambda b, sl, ex: (0, 0)),
            pl.BlockSpec((2, 1), lambda b, sl, ex: (0, 0)),
            pl.BlockSpec((C, H), lambda b, sl, ex: (0, 0)),
            pl.BlockSpec((C, 1), lambda b, sl, ex: (0, 0)),
        ],
        out_specs=pl.BlockSpec((1, C, 1), lambda b, sl, ex: (b, 0, 0)),
    )
    out = pl.pallas_call(
        _tc_body,
        grid_spec=grid_spec,
        out_shape=jax.ShapeDtypeStruct((B, C, 1), jnp.float32),
    )(
        step_limit, exit_index,
        tok_embt, row(starts), row(ends), col(tidx), col(fidx),
        w_in.T, w_h.T, b_h.reshape(H, 1), w_branch.T,
        b_branch.reshape(2, 1), w_out.T, b_out.reshape(C, 1),
    )
    return out.reshape(B, C)


def kernel(tokens, node_token_span_starts, node_token_span_ends,
           edge_sources, edge_dests, edge_types, true_branch_nodes,
           false_branch_nodes, exit_index, step_limit, embed, W_in, W_h,
           b_h, W_branch, b_branch, W_out, b_out):
    i32 = jnp.int32
    tokens = tokens.astype(i32)
    tok_emb = _sc_gather(embed, tokens.reshape(-1)).reshape(B, T, H)
    return _tc_main(
        tok_emb,
        node_token_span_starts.astype(i32),
        node_token_span_ends.astype(i32),
        true_branch_nodes.astype(i32),
        false_branch_nodes.astype(i32),
        step_limit.astype(i32),
        exit_index.astype(i32),
        W_in, W_h, b_h, W_branch, b_branch, W_out, b_out,
    )


# speedup vs baseline: 1.0803x; 1.0019x over previous
import functools

import jax
import jax.numpy as jnp
from jax import lax
from jax.experimental import pallas as pl
from jax.experimental.pallas import tpu as pltpu
from jax.experimental.pallas import tpu_sc as plsc

B = 32
T = 512
N = 128
H = 256
C = 27
STEPS = 64


def _sc_gather(table, idx):
    info = plsc.get_sparse_core_info()
    nw = info.num_cores * info.num_subcores
    total = idx.shape[0]
    per_w = total // nw
    chunk = 256
    n_chunks = per_w // chunk
    h = table.shape[1]
    mesh = plsc.VectorSubcoreMesh(core_axis_name="c", subcore_axis_name="s")

    @functools.partial(
        pl.kernel,
        mesh=mesh,
        out_type=jax.ShapeDtypeStruct((total, h), jnp.float32),
        scratch_types=[
            pltpu.VMEM((chunk,), jnp.int32),
            pltpu.VMEM((chunk, h), jnp.float32),
            pltpu.SemaphoreType.DMA,
        ],
    )
    def k(table_hbm, idx_hbm, out_hbm, idx_v, rows_v, sem):
        wid = lax.axis_index("s") * info.num_cores + lax.axis_index("c")
        base = wid * per_w

        def body(i, carry):
            off = base + i * chunk
            pltpu.sync_copy(idx_hbm.at[pl.ds(off, chunk)], idx_v)
            pltpu.async_copy(table_hbm.at[idx_v], rows_v, sem).wait()
            pltpu.sync_copy(rows_v, out_hbm.at[pl.ds(off, chunk)])
            return carry

        lax.fori_loop(0, n_chunks, body, 0)

    return k(table, idx)


def _dot(a, b):
    return jnp.dot(a, b, preferred_element_type=jnp.float32)


def _bdot(a, b):
    return jnp.dot(a.astype(jnp.bfloat16), b.astype(jnp.bfloat16),
                   preferred_element_type=jnp.float32)


def _fused_body(msl_ref, te_ref, st_ref, en_ref, sl_ref, exr_ref,
                ti_ref, fi_ref, wit_ref, bh_ref,
                wht_ref, wbt1_ref, bb_ref, wot_ref, bo_ref, out_ref):
    bf16 = jnp.bfloat16
    BN = B * N

    post = lax.broadcasted_iota(jnp.int32, (T, N), 0)
    witb = wit_ref[...].astype(bf16)
    xparts = []
    for b in range(B):
        s_row = st_ref[b]
        e_row = en_ref[b]
        maskt = jnp.where((post >= s_row) & (post <= e_row),
                          1.0, 0.0).astype(bf16)
        cnt = jnp.maximum(e_row - s_row + 1, 1).astype(jnp.float32)
        node_t = lax.dot_general(
            te_ref[b].astype(bf16), maskt, (((0,), (0,)), ((), ())),
            preferred_element_type=jnp.float32,
        ) / cnt
        xparts.append(
            jnp.dot(witb, node_t.astype(bf16),
                    preferred_element_type=jnp.float32))
    x_int = jnp.concatenate(xparts, axis=1) + bh_ref[...]
    sl_row = sl_ref[...]
    wht = wht_ref[...].astype(bf16)
    wbt1 = wbt1_ref[...].astype(bf16)
    bb = bb_ref[...]

    lanes = lax.broadcasted_iota(jnp.int32, (N, N), 1)
    subl = lax.broadcasted_iota(jnp.int32, (N, N), 0)
    ident = jnp.where(lanes == subl, 1.0, 0.0)
    ohts = [jnp.where(lanes == ti_ref[b], 1.0, 0.0) for b in range(B)]
    ohfs = [jnp.where(lanes == fi_ref[b], 1.0, 0.0) for b in range(B)]

    lane_mod = lax.broadcasted_iota(jnp.int32, (1, BN), 1) % N
    p0 = jnp.where(lane_mod == 0, 1.0, 0.0)
    h0 = jnp.zeros((H, BN), bf16)
    ones_row = jnp.ones((1, BN), jnp.float32)

    def step(t, carry):
        p, htb = carry
        ht_new = jnp.tanh(x_int + jnp.dot(
            wht, htb, preferred_element_type=jnp.float32))
        ht1b = jnp.concatenate([ht_new, ones_row], axis=0).astype(bf16)
        lgt = jnp.dot(wbt1, ht1b, preferred_element_type=jnp.float32) + bb
        mx = jnp.max(lgt, axis=0, keepdims=True)
        elg = jnp.exp(lgt - mx)
        den = jnp.sum(elg, axis=0, keepdims=True)
        pt = p * (elg[0:1, :] / den)
        pf = p * (elg[1:2, :] / den)
        parts = []
        for b in range(B):
            s = slice(b * N, (b + 1) * N)
            ptc = jnp.sum(ident * pt[:, s], axis=1, keepdims=True)
            pfc = jnp.sum(ident * pf[:, s], axis=1, keepdims=True)
            m = (ohts[b] * ptc + ohfs[b] * pfc).astype(bf16)
            parts.append(jnp.dot(ht1b[:, s], m,
                                 preferred_element_type=jnp.float32))
        r = jnp.concatenate(parts, axis=1)
        p_next = r[H:H + 1, :]
        h_next = r[0:H, :] / jnp.maximum(p_next, 1e-6)
        act = t < sl_row
        p = jnp.where(act, p_next, p)
        htb = jnp.where(act, h_next.astype(bf16), htb)
        return (p, htb)

    msl = msl_ref[0]
    p, htb = lax.fori_loop(0, msl, step, (p0, h0))

    hx = htb.astype(jnp.float32) * exr_ref[...]
    cols = [jnp.sum(hx[:, b * N:(b + 1) * N], axis=1, keepdims=True)
            for b in range(B)]
    hsel = jnp.concatenate(cols, axis=1)
    out_ref[...] = _dot(wot_ref[...], hsel) + bo_ref[...]


def _tc_main(tok_emb, starts, ends, tidx, fidx, step_limit, exit_index,
             w_in, w_h, b_h, w_branch, b_branch, w_out, b_out):
    row = lambda a: a.reshape(B, 1, N)
    BN = B * N

    sl_row = jnp.repeat(step_limit, N).reshape(1, BN)
    exr = (jnp.arange(N, dtype=jnp.int32)[None, :]
           == exit_index[:, None]).astype(jnp.float32).reshape(1, BN)
    msl = jnp.max(step_limit).reshape(1)

    grid_spec = pltpu.PrefetchScalarGridSpec(
        num_scalar_prefetch=1,
        grid=(1,),
        in_specs=[
            pl.BlockSpec((B, T, H), lambda g, msl_r: (0, 0, 0)),
            pl.BlockSpec((B, 1, N), lambda g, msl_r: (0, 0, 0)),
            pl.BlockSpec((B, 1, N), lambda g, msl_r: (0, 0, 0)),
            pl.BlockSpec((1, BN), lambda g, msl_r: (0, 0)),
            pl.BlockSpec((1, BN), lambda g, msl_r: (0, 0)),
            pl.BlockSpec((B, N, 1), lambda g, msl_r: (0, 0, 0)),
            pl.BlockSpec((B, N, 1), lambda g, msl_r: (0, 0, 0)),
            pl.BlockSpec((H, H), lambda g, msl_r: (0, 0)),
            pl.BlockSpec((H, 1), lambda g, msl_r: (0, 0)),
            pl.BlockSpec((H, H), lambda g, msl_r: (0, 0)),
            pl.BlockSpec((2, H + 1), lambda g, msl_r: (0, 0)),
            pl.BlockSpec((2, 1), lambda g, msl_r: (0, 0)),
            pl.BlockSpec((C, H), lambda g, msl_r: (0, 0)),
            pl.BlockSpec((C, 1), lambda g, msl_r: (0, 0)),
        ],
        out_specs=pl.BlockSpec((C, B), lambda g, msl_r: (0, 0)),
    )
    out = pl.pallas_call(
        _fused_body,
        grid_spec=grid_spec,
        out_shape=jax.ShapeDtypeStruct((C, B), jnp.float32),
    )(
        msl, tok_emb, row(starts), row(ends), sl_row, exr,
        tidx.reshape(B, N, 1), fidx.reshape(B, N, 1),
        w_in.T, b_h.reshape(H, 1),
        w_h.T,
        jnp.concatenate([w_branch.T, jnp.zeros((2, 1), jnp.float32)], axis=1),
        b_branch.reshape(2, 1),
        w_out.T, b_out.reshape(C, 1),
    )
    return out.T


def kernel(tokens, node_token_span_starts, node_token_span_ends,
           edge_sources, edge_dests, edge_types, true_branch_nodes,
           false_branch_nodes, exit_index, step_limit, embed, W_in, W_h,
           b_h, W_branch, b_branch, W_out, b_out):
    i32 = jnp.int32
    tokens = tokens.astype(i32)
    tok_emb = _sc_gather(embed, tokens.reshape(-1)).reshape(B, T, H)
    return _tc_main(
        tok_emb,
        node_token_span_starts.astype(i32),
        node_token_span_ends.astype(i32),
        true_branch_nodes.astype(i32),
        false_branch_nodes.astype(i32),
        step_limit.astype(i32),
        exit_index.astype(i32),
        W_in, W_h, b_h, W_branch, b_branch, W_out, b_out,
    )
